# Initial kernel scaffold; baseline (speedup 1.0000x reference)
import jax
import jax.numpy as jnp
from jax.experimental import pallas as pl


def kernel(indices, weight):
    raise NotImplementedError("write your pallas kernel here")


# speedup vs baseline: 1.2651x; 1.2651x over previous
import functools

import jax
import jax.numpy as jnp
from jax import lax
from jax.experimental import pallas as pl
from jax.experimental.pallas import tpu as pltpu
from jax.experimental.pallas import tpu_sc as plsc

NUM_SHAPES = 100000
LATENT_DIM = 256
BATCH = 16384

_NC = 2
_NS = 16
_NW = _NC * _NS
_BPW = BATCH // _NW
_CHUNK = 128
_NCHUNK = _BPW // _CHUNK


def _make_gather():
    mesh = plsc.VectorSubcoreMesh(core_axis_name="c", subcore_axis_name="s")

    @functools.partial(
        pl.kernel,
        mesh=mesh,
        out_type=jax.ShapeDtypeStruct((BATCH, LATENT_DIM), jnp.float32),
        scratch_types=[
            pltpu.VMEM((_NCHUNK, _CHUNK), jnp.int32),
            pltpu.VMEM((2, _CHUNK, LATENT_DIM), jnp.float32),
            pltpu.SemaphoreType.DMA,
        ],
    )
    def gather_kernel(idx_hbm, table_hbm, out_hbm, idx_v, rows_v, sem):
        wid = lax.axis_index("s") * _NC + lax.axis_index("c")
        base = wid * _BPW
        for c in range(_NCHUNK):
            pltpu.sync_copy(
                idx_hbm.at[pl.ds(base + c * _CHUNK, _CHUNK)], idx_v.at[c]
            )
        for c in range(_NCHUNK):
            buf = rows_v.at[c % 2]
            pltpu.async_copy(table_hbm.at[idx_v.at[c]], buf, sem).wait()
            pltpu.sync_copy(
                buf, out_hbm.at[pl.ds(base + c * _CHUNK, _CHUNK)]
            )

    return gather_kernel


_gather = _make_gather()


@jax.jit
def kernel(indices, weight):
    return _gather(indices, weight)


# speedup vs baseline: 1.3856x; 1.0952x over previous
import functools

import jax
import jax.numpy as jnp
from jax import lax
from jax.experimental import pallas as pl
from jax.experimental.pallas import tpu as pltpu
from jax.experimental.pallas import tpu_sc as plsc

NUM_SHAPES = 100000
LATENT_DIM = 256
BATCH = 16384

_NC = 2
_NS = 16
_NW = _NC * _NS
_BPW = BATCH // _NW
_CHUNK = 128
_NCHUNK = _BPW // _CHUNK


def _make_gather():
    mesh = plsc.VectorSubcoreMesh(core_axis_name="c", subcore_axis_name="s")

    @functools.partial(
        pl.kernel,
        mesh=mesh,
        out_type=jax.ShapeDtypeStruct((BATCH, LATENT_DIM), jnp.float32),
        scratch_types=[
            pltpu.VMEM((_NCHUNK, _CHUNK), jnp.int32),
            pltpu.VMEM((3, _CHUNK, LATENT_DIM), jnp.float32),
            pltpu.SemaphoreType.DMA,
            pltpu.SemaphoreType.DMA,
            pltpu.SemaphoreType.DMA,
            pltpu.SemaphoreType.DMA,
            pltpu.SemaphoreType.DMA,
            pltpu.SemaphoreType.DMA,
        ],
    )
    def gather_kernel(
        idx_hbm, table_hbm, out_hbm, idx_v, rows_v,
        g0, g1, g2, o0, o1, o2,
    ):
        gsem = [g0, g1, g2]
        osem = [o0, o1, o2]
        wid = lax.axis_index("s") * _NC + lax.axis_index("c")
        base = wid * _BPW
        for c in range(_NCHUNK):
            pltpu.sync_copy(
                idx_hbm.at[pl.ds(base + c * _CHUNK, _CHUNK)], idx_v.at[c]
            )
        gathers = [None] * _NCHUNK
        outs = [None] * _NCHUNK
        for c in range(min(3, _NCHUNK)):
            gathers[c] = pltpu.async_copy(
                table_hbm.at[idx_v.at[c]], rows_v.at[c % 3], gsem[c % 3]
            )
        for c in range(_NCHUNK):
            gathers[c].wait()
            outs[c] = pltpu.async_copy(
                rows_v.at[c % 3],
                out_hbm.at[pl.ds(base + c * _CHUNK, _CHUNK)],
                osem[c % 3],
            )
            nxt = c + 3
            if nxt < _NCHUNK:
                outs[nxt - 3].wait()
                gathers[nxt] = pltpu.async_copy(
                    table_hbm.at[idx_v.at[nxt]], rows_v.at[nxt % 3], gsem[nxt % 3]
                )
        for c in range(max(0, _NCHUNK - 3), _NCHUNK):
            outs[c].wait()

    return gather_kernel


_gather = _make_gather()


@jax.jit
def kernel(indices, weight):
    return _gather(indices, weight)
